# Initial kernel scaffold; baseline (speedup 1.0000x reference)
import jax
import jax.numpy as jnp
from jax.experimental import pallas as pl


def kernel(x, edge_index, W0, b0, W1, b1, W2, b2, gw0, gb0, gw1, gb1, gw2, gb2, gw3, gb3, lpw0, lpb0, lpw1, lpb1, lpw2, lpb2, cw, cb, h0, beta):
    raise NotImplementedError("write your pallas kernel here")


# speedup vs baseline: 8.2072x; 8.2072x over previous
import functools

import jax
import jax.numpy as jnp
from jax import lax
from jax.experimental import pallas as pl
from jax.experimental.pallas import tpu as pltpu
from jax.experimental.pallas import tpu_sc as plsc

N = 10000
DH = 128
DOUT = 64
E = 320000

NC = 2
NS = 16
NW = NC * NS
BLK = 128
NBLK = -(-E // (NW * BLK))
E_PAD = NW * BLK * NBLK
N_PAD = N + 112
ROWS_PER_TILE = N_PAD // NS

_mesh = plsc.VectorSubcoreMesh(core_axis_name="c", subcore_axis_name="s")


@functools.partial(
    pl.kernel,
    out_type=jax.ShapeDtypeStruct((NC, N_PAD, DH), jnp.float32),
    mesh=_mesh,
    scratch_types=[
        pltpu.VMEM((BLK,), jnp.int32),
        pltpu.VMEM((BLK, DH), jnp.float32),
        pltpu.VMEM_SHARED((N_PAD, DH), jnp.float32),
    ],
)
def _sc_degree(dst_hbm, zeros_hbm, ones_hbm, out_hbm, idx_v, ones_v, acc_sh):
    c = lax.axis_index("c")
    s = lax.axis_index("s")
    wid = c * NS + s
    pltpu.sync_copy(ones_hbm, ones_v)
    r0 = s * ROWS_PER_TILE
    pltpu.sync_copy(zeros_hbm.at[pl.ds(r0, ROWS_PER_TILE)],
                    acc_sh.at[pl.ds(r0, ROWS_PER_TILE)])
    plsc.subcore_barrier()

    def blk(j, carry):
        base = (wid * NBLK + j) * BLK
        pltpu.sync_copy(dst_hbm.at[pl.ds(base, BLK)], idx_v)
        pltpu.sync_copy(ones_v, acc_sh.at[idx_v], add=True)
        return carry

    lax.fori_loop(0, NBLK, blk, 0)
    plsc.subcore_barrier()
    pltpu.sync_copy(acc_sh.at[pl.ds(r0, ROWS_PER_TILE)],
                    out_hbm.at[c, pl.ds(r0, ROWS_PER_TILE)])


@functools.partial(
    pl.kernel,
    out_type=jax.ShapeDtypeStruct((NC, N_PAD, DH), jnp.float32),
    mesh=_mesh,
    scratch_types=[
        pltpu.VMEM((BLK,), jnp.int32),
        pltpu.VMEM((BLK,), jnp.int32),
        pltpu.VMEM((BLK, DH), jnp.float32),
        pltpu.VMEM_SHARED((N_PAD, DH), jnp.float32),
        pltpu.SemaphoreType.DMA,
    ],
)
def _sc_scatter(hp_hbm, src_hbm, dst_hbm, zeros_hbm, out_hbm,
                sidx, didx, rows_v, acc_sh, sem):
    c = lax.axis_index("c")
    s = lax.axis_index("s")
    wid = c * NS + s
    r0 = s * ROWS_PER_TILE
    pltpu.sync_copy(zeros_hbm.at[pl.ds(r0, ROWS_PER_TILE)],
                    acc_sh.at[pl.ds(r0, ROWS_PER_TILE)])
    plsc.subcore_barrier()

    def blk(j, carry):
        base = (wid * NBLK + j) * BLK
        pltpu.sync_copy(src_hbm.at[pl.ds(base, BLK)], sidx)
        pltpu.sync_copy(dst_hbm.at[pl.ds(base, BLK)], didx)
        pltpu.async_copy(hp_hbm.at[sidx], rows_v, sem).wait()
        pltpu.sync_copy(rows_v, acc_sh.at[didx], add=True)
        return carry

    lax.fori_loop(0, NBLK, blk, 0)
    plsc.subcore_barrier()
    pltpu.sync_copy(acc_sh.at[pl.ds(r0, ROWS_PER_TILE)],
                    out_hbm.at[c, pl.ds(r0, ROWS_PER_TILE)])


def _dis_from_degp(degp_ref):
    deg = degp_ref[0][:N, 0:1] + degp_ref[1][:N, 0:1] + 1.0
    return lax.rsqrt(deg)


def _pool(x, gwt, gb, lp, lpb):
    logit = jnp.sum(x * gwt, axis=1, keepdims=True) + gb
    e = jnp.exp(logit - jnp.max(logit))
    pool = jnp.sum(x * e, axis=0, keepdims=True) / jnp.sum(e)
    return jnp.dot(pool, lp, preferred_element_type=jnp.float32) + lpb


def _tc0_body(x_ref, degp_ref, W_ref, gwt_ref, gb_ref, lp_ref, lpb_ref,
              hp_ref, pool_ref):
    x = x_ref[...]
    dis = _dis_from_degp(degp_ref)
    pool_ref[...] = _pool(x, gwt_ref[...], gb_ref[...], lp_ref[...], lpb_ref[...])
    hp_ref[...] = jnp.dot(x, W_ref[...], preferred_element_type=jnp.float32) * dis


def _tcmid_body(aggp_ref, hp_prev_ref, degp_ref, b_ref, W_ref,
                gwt_ref, gb_ref, lp_ref, lpb_ref, hp_ref, pool_ref):
    dis = _dis_from_degp(degp_ref)
    agg = aggp_ref[0][:N, :] + aggp_ref[1][:N, :] + hp_prev_ref[...]
    x = agg * dis + b_ref[...]
    pool_ref[...] = _pool(x, gwt_ref[...], gb_ref[...], lp_ref[...], lpb_ref[...])
    hp_ref[...] = jnp.dot(x, W_ref[...], preferred_element_type=jnp.float32) * dis


def _tcfin_body(aggp_ref, hp_prev_ref, degp_ref, b_ref,
                gwt_ref, gb_ref, cw_ref, cb_ref,
                p0_ref, p1_ref, p2_ref, h0_ref, beta_ref, risk_ref):
    dis = _dis_from_degp(degp_ref)
    agg = aggp_ref[0][:N, :] + aggp_ref[1][:N, :] + hp_prev_ref[...]
    x = agg * dis + b_ref[...]
    p3 = _pool(x, gwt_ref[...], gb_ref[...], cw_ref[...], cb_ref[...])
    out = (p0_ref[...] + p1_ref[...] + p2_ref[...] + p3) * 0.25
    val = jnp.sum(out * beta_ref[...])
    risk_ref[...] = jnp.exp(h0_ref[...] + val)


def _tc_stage0(x, degp, W, gwt, gb, lp, lpb):
    return pl.pallas_call(
        _tc0_body,
        out_shape=[jax.ShapeDtypeStruct((N, DH), jnp.float32),
                   jax.ShapeDtypeStruct((1, DOUT), jnp.float32)],
    )(x, degp, W, gwt, gb, lp, lpb)


def _tc_stage(aggp, hp_prev, degp, b, W, gwt, gb, lp, lpb):
    return pl.pallas_call(
        _tcmid_body,
        out_shape=[jax.ShapeDtypeStruct((N, DH), jnp.float32),
                   jax.ShapeDtypeStruct((1, DOUT), jnp.float32)],
    )(aggp, hp_prev, degp, b, W, gwt, gb, lp, lpb)


def _tc_final(aggp, hp_prev, degp, b, gwt, gb, cw, cb, p0, p1, p2, h0, beta):
    return pl.pallas_call(
        _tcfin_body,
        out_shape=jax.ShapeDtypeStruct((1, 1), jnp.float32),
    )(aggp, hp_prev, degp, b, gwt, gb, cw, cb, p0, p1, p2, h0, beta)


def kernel(x, edge_index, W0, b0, W1, b1, W2, b2, gw0, gb0, gw1, gb1,
           gw2, gb2, gw3, gb3, lpw0, lpb0, lpw1, lpb1, lpw2, lpb2,
           cw, cb, h0, beta):
    src = edge_index[0]
    dst = edge_index[1]
    pad = E_PAD - E
    srcp = jnp.concatenate([src, jnp.zeros((pad,), jnp.int32)])
    dstp = jnp.concatenate([dst, jnp.full((pad,), N, jnp.int32)])
    zrow = jnp.zeros((N_PAD, DH), jnp.float32)
    ones_blk = jnp.ones((BLK, DH), jnp.float32)

    degp = _sc_degree(dstp, zrow, ones_blk)

    hp0, p0 = _tc_stage0(x, degp, W0, gw0.reshape(1, DH), gb0.reshape(1, 1),
                         lpw0, lpb0.reshape(1, DOUT))
    agg0 = _sc_scatter(hp0, srcp, dstp, zrow)
    hp1, p1 = _tc_stage(agg0, hp0, degp, b0.reshape(1, DH), W1,
                        gw1.reshape(1, DH), gb1.reshape(1, 1),
                        lpw1, lpb1.reshape(1, DOUT))
    agg1 = _sc_scatter(hp1, srcp, dstp, zrow)
    hp2, p2 = _tc_stage(agg1, hp1, degp, b1.reshape(1, DH), W2,
                        gw2.reshape(1, DH), gb2.reshape(1, 1),
                        lpw2, lpb2.reshape(1, DOUT))
    agg2 = _sc_scatter(hp2, srcp, dstp, zrow)
    risk = _tc_final(agg2, hp2, degp, b2.reshape(1, DH),
                     gw3.reshape(1, DH), gb3.reshape(1, 1), cw,
                     cb.reshape(1, DOUT), p0, p1, p2,
                     h0.reshape(1, 1), beta.reshape(1, DOUT))
    return risk.reshape(1)
